# baseline (speedup 1.0000x reference)
import functools

import jax
import jax.numpy as jnp
from jax import lax
from jax.experimental import pallas as pl
from jax.experimental.pallas import tpu as pltpu
from jax.experimental.pallas import tpu_sc as plsc

F = 32
BATCH = 16384
NU = 1000000
NI = 100000

NC = 2
NS = 16
NW = NC * NS
BPW = BATCH // NW
CHUNK = 128
NCHUNK = BPW // CHUNK
L = 16

NBU = NU // 128
UTAIL = NU - NBU * 128
NBI = NI // 128
ITAIL = NI - NBI * 128
UPK = (NBU + 1) * F * 128
IPK = (NBI + 1) * F * 128

UB_PER = NBU // NW
UB_LEFT = NBU - UB_PER * NW
IB_PER = NBI // NW
IB_LEFT = NBI - IB_PER * NW


def _repack_body(uft_hbm, ift_hbm, utail_hbm, itail_hbm,
                 upk_hbm, ipk_hbm, sem):
    wid = lax.axis_index("s") * NC + lax.axis_index("c")

    def ucopy(blk, width):
        src = uft_hbm.at[:, pl.ds(pl.multiple_of(blk * 128, 128), width)]
        dst = upk_hbm.at[pl.ds(pl.multiple_of(blk * F, 8), F),
                         pl.ds(0, width)]
        return pltpu.make_async_copy(src, dst, sem)

    def icopy(blk, width):
        src = ift_hbm.at[:, pl.ds(pl.multiple_of(blk * 128, 128), width)]
        dst = ipk_hbm.at[pl.ds(pl.multiple_of(blk * F, 8), F),
                         pl.ds(0, width)]
        return pltpu.make_async_copy(src, dst, sem)

    def fire_user(n, _):
        ucopy(wid * UB_PER + n, 128).start()
        return 0

    def fire_item(n, _):
        icopy(wid * IB_PER + n, 128).start()
        return 0

    lax.fori_loop(0, UB_PER, fire_user, 0)
    lax.fori_loop(0, IB_PER, fire_item, 0)
    @pl.when(wid < UB_LEFT)
    def _():
        ucopy(NW * UB_PER + wid, 128).start()

    @pl.when(wid == 4)
    def _():
        pltpu.make_async_copy(
            utail_hbm, upk_hbm.at[pl.ds(NBU * F, F), :], sem).start()

    @pl.when(jnp.logical_and(wid >= 8, wid < 8 + IB_LEFT))
    def _():
        icopy(NW * IB_PER + (wid - 8), 128).start()

    @pl.when(wid == 21)
    def _():
        pltpu.make_async_copy(
            itail_hbm, ipk_hbm.at[pl.ds(NBI * F, F), :], sem).start()

    def drain_user(n, _):
        ucopy(wid * UB_PER + n, 128).wait()
        return 0

    def drain_item(n, _):
        icopy(wid * IB_PER + n, 128).wait()
        return 0

    lax.fori_loop(0, UB_PER, drain_user, 0)
    lax.fori_loop(0, IB_PER, drain_item, 0)
    @pl.when(wid < UB_LEFT)
    def _():
        ucopy(NW * UB_PER + wid, 128).wait()

    @pl.when(wid == 4)
    def _():
        pltpu.make_async_copy(
            utail_hbm, upk_hbm.at[pl.ds(NBU * F, F), :], sem).wait()

    @pl.when(jnp.logical_and(wid >= 8, wid < 8 + IB_LEFT))
    def _():
        icopy(NW * IB_PER + (wid - 8), 128).wait()

    @pl.when(wid == 21)
    def _():
        pltpu.make_async_copy(
            itail_hbm, ipk_hbm.at[pl.ds(NBI * F, F), :], sem).wait()


def _gather_body(user_hbm, item_hbm, upk_hbm, ipk_hbm, out_hbm,
                 uidx, iidx, ufidx, ifidx, ut, vt, oloc, sem):
    wid = lax.axis_index("s") * NC + lax.axis_index("c")
    base = wid * BPW

    for j in range(NCHUNK):
        pltpu.sync_copy(user_hbm.at[pl.ds(base + j * CHUNK, CHUNK)], uidx.at[j])
        pltpu.sync_copy(item_hbm.at[pl.ds(base + j * CHUNK, CHUNK)], iidx.at[j])

    def build(f, _):
        for j in range(NCHUNK):
            for v in range(CHUNK // L):
                r = uidx[j, pl.ds(v * L, L)]
                ufidx[f, j, pl.ds(v * L, L)] = (
                    (r >> 7) * (F * 128) + f * 128 + (r & 127))
                s = iidx[j, pl.ds(v * L, L)]
                ifidx[f, j, pl.ds(v * L, L)] = (
                    (s >> 7) * (F * 128) + f * 128 + (s & 127))
        return 0

    lax.fori_loop(0, F, build, 0)

    def fire(f, _):
        for j in range(NCHUNK):
            pltpu.async_copy(upk_hbm.at[ufidx.at[f, j]],
                             ut.at[f, pl.ds(j * CHUNK, CHUNK)], sem)
            pltpu.async_copy(ipk_hbm.at[ifidx.at[f, j]],
                             vt.at[f, pl.ds(j * CHUNK, CHUNK)], sem)
        return 0

    lax.fori_loop(0, F, fire, 0)

    def drain(f, _):
        for j in range(NCHUNK):
            pltpu.make_async_copy(upk_hbm.at[ufidx.at[f, j]],
                                  ut.at[f, pl.ds(j * CHUNK, CHUNK)],
                                  sem).wait()
            pltpu.make_async_copy(ipk_hbm.at[ifidx.at[f, j]],
                                  vt.at[f, pl.ds(j * CHUNK, CHUNK)],
                                  sem).wait()
        return 0

    lax.fori_loop(0, F, drain, 0)

    def reduce_group(g, _):
        b0 = g * L
        acc = ut[0, pl.ds(b0, L)] * vt[0, pl.ds(b0, L)]
        for f in range(1, F):
            acc = acc + ut[f, pl.ds(b0, L)] * vt[f, pl.ds(b0, L)]
        oloc[pl.ds(b0, L)] = acc
        return 0

    lax.fori_loop(0, BPW // L, reduce_group, 0)

    pltpu.sync_copy(oloc, out_hbm.at[pl.ds(base, BPW)])


@jax.jit
def _mf_scores(user, item, user_factors, item_factors):
    mesh = plsc.VectorSubcoreMesh(core_axis_name="c", subcore_axis_name="s")
    repack = functools.partial(
        pl.kernel,
        out_type=(jax.ShapeDtypeStruct((UPK // 128, 128), jnp.float32),
                  jax.ShapeDtypeStruct((IPK // 128, 128), jnp.float32)),
        mesh=mesh,
        compiler_params=pltpu.CompilerParams(needs_layout_passes=False),
        scratch_types=[pltpu.SemaphoreType.DMA],
    )(_repack_body)
    utail = jnp.pad(user_factors[NBU * 128:].T, ((0, 0), (0, 128 - UTAIL)))
    itail = jnp.pad(item_factors[NBI * 128:].T, ((0, 0), (0, 128 - ITAIL)))
    upk2, ipk2 = repack(user_factors.T, item_factors.T, utail, itail)
    upk = upk2.reshape(UPK)
    ipk = ipk2.reshape(IPK)

    gather = functools.partial(
        pl.kernel,
        out_type=jax.ShapeDtypeStruct((BATCH,), jnp.float32),
        mesh=mesh,
        compiler_params=pltpu.CompilerParams(
            needs_layout_passes=False, use_tc_tiling_on_sc=False),
        scratch_types=[
            pltpu.VMEM((NCHUNK, CHUNK), jnp.int32),
            pltpu.VMEM((NCHUNK, CHUNK), jnp.int32),
            pltpu.VMEM((F, NCHUNK, CHUNK), jnp.int32),
            pltpu.VMEM((F, NCHUNK, CHUNK), jnp.int32),
            pltpu.VMEM((F, BPW), jnp.float32),
            pltpu.VMEM((F, BPW), jnp.float32),
            pltpu.VMEM((BPW,), jnp.float32),
            pltpu.SemaphoreType.DMA,
        ],
    )(_gather_body)
    return gather(user, item, upk, ipk)


def kernel(user, item, user_factors, item_factors):
    return _mf_scores(user.astype(jnp.int32), item.astype(jnp.int32),
                      user_factors, item_factors)


# speedup vs baseline: 23.8229x; 23.8229x over previous
import functools

import jax
import jax.numpy as jnp
from jax import lax
from jax.experimental import pallas as pl
from jax.experimental.pallas import tpu as pltpu
from jax.experimental.pallas import tpu_sc as plsc

F = 32
BATCH = 16384
NU = 1000000
NI = 100000

NC = 2
NS = 16
NW = NC * NS
BPW = BATCH // NW
CHUNK = 128
NCHUNK = BPW // CHUNK
L = 16

NBU = NU // 128
UTAIL = NU - NBU * 128
NBI = NI // 128
ITAIL = NI - NBI * 128

G = 8
GROWS = G * F
NGU = (NBU // G) * G // G
NGI = (NBI // G) * G // G
UROWS = (NGU + 1) * GROWS
IROWS = (NGI + 1) * GROWS
UPK = UROWS * 128
IPK = IROWS * 128
NITER_U = 31
NITER_I = 4


def _repack_body(uft_hbm, ift_hbm, utail_hbm, itail_hbm,
                 upk_hbm, ipk_hbm, slab, sem_r, sem_w):
    wid = lax.axis_index("s") * NC + lax.axis_index("c")

    def pump(src_hbm, tail_hbm, dst_hbm, ngrp, niter):
        def rd(n):
            grp = wid + NW * n
            p = n & 1
            for b in range(G):
                dstb = slab.at[p, pl.ds(b * F, F), :]
                @pl.when(grp < ngrp)
                def _():
                    pltpu.make_async_copy(
                        src_hbm.at[:, pl.ds(
                            pl.multiple_of((grp * G + b) * 128, 128), 128)],
                        dstb, sem_r).start()
                @pl.when(grp == ngrp)
                def _():
                    pltpu.make_async_copy(
                        tail_hbm.at[:, pl.ds(b * 128, 128)], dstb,
                        sem_r).start()
        def rd_wait(n):
            p = n & 1
            for b in range(G):
                pltpu.make_async_copy(
                    tail_hbm.at[:, pl.ds(b * 128, 128)],
                    slab.at[p, pl.ds(b * F, F), :], sem_r).wait()

        def wr(n):
            grp = wid + NW * n
            return pltpu.make_async_copy(
                slab.at[n & 1],
                dst_hbm.at[pl.ds(pl.multiple_of(grp * GROWS, 8), GROWS), :],
                sem_w)

        def step(n, _):
            grp = wid + NW * n
            @pl.when(grp <= ngrp)
            def _():
                rd_wait(n)
                wr(n).start()
            @pl.when(wid + NW * (n + 1) <= ngrp)
            def _():
                rd(n + 1)
            @pl.when(grp <= ngrp)
            def _():
                wr(n).wait()
            return 0

        @pl.when(wid <= ngrp)
        def _():
            rd(0)
        lax.fori_loop(0, niter, step, 0)

    pump(uft_hbm, utail_hbm, upk_hbm, NGU, NITER_U)
    pump(ift_hbm, itail_hbm, ipk_hbm, NGI, NITER_I)


def _gather_body(user_hbm, item_hbm, upk_hbm, ipk_hbm, out_hbm,
                 uidx, iidx, ufidx, ifidx, ut, vt, oloc, sem):
    wid = lax.axis_index("s") * NC + lax.axis_index("c")
    base = wid * BPW

    for j in range(NCHUNK):
        pltpu.sync_copy(user_hbm.at[pl.ds(base + j * CHUNK, CHUNK)], uidx.at[j])
        pltpu.sync_copy(item_hbm.at[pl.ds(base + j * CHUNK, CHUNK)], iidx.at[j])

    def build(f, _):
        for j in range(NCHUNK):
            for v in range(CHUNK // L):
                r = uidx[j, pl.ds(v * L, L)]
                ufidx[f, j, pl.ds(v * L, L)] = (
                    (r >> 7) * (F * 128) + f * 128 + (r & 127))
                s = iidx[j, pl.ds(v * L, L)]
                ifidx[f, j, pl.ds(v * L, L)] = (
                    (s >> 7) * (F * 128) + f * 128 + (s & 127))
        return 0

    lax.fori_loop(0, F, build, 0)

    def fire(f, _):
        for j in range(NCHUNK):
            pltpu.async_copy(upk_hbm.at[ufidx.at[f, j]],
                             ut.at[f, pl.ds(j * CHUNK, CHUNK)], sem)
            pltpu.async_copy(ipk_hbm.at[ifidx.at[f, j]],
                             vt.at[f, pl.ds(j * CHUNK, CHUNK)], sem)
        return 0

    lax.fori_loop(0, F, fire, 0)

    def drain(f, _):
        for j in range(NCHUNK):
            pltpu.make_async_copy(upk_hbm.at[ufidx.at[f, j]],
                                  ut.at[f, pl.ds(j * CHUNK, CHUNK)],
                                  sem).wait()
            pltpu.make_async_copy(ipk_hbm.at[ifidx.at[f, j]],
                                  vt.at[f, pl.ds(j * CHUNK, CHUNK)],
                                  sem).wait()
        return 0

    lax.fori_loop(0, F, drain, 0)

    def reduce_group(g, _):
        b0 = g * L
        acc = ut[0, pl.ds(b0, L)] * vt[0, pl.ds(b0, L)]
        for f in range(1, F):
            acc = acc + ut[f, pl.ds(b0, L)] * vt[f, pl.ds(b0, L)]
        oloc[pl.ds(b0, L)] = acc
        return 0

    lax.fori_loop(0, BPW // L, reduce_group, 0)

    pltpu.sync_copy(oloc, out_hbm.at[pl.ds(base, BPW)])


@jax.jit
def _mf_scores(user, item, user_factors, item_factors):
    mesh = plsc.VectorSubcoreMesh(core_axis_name="c", subcore_axis_name="s")
    repack = functools.partial(
        pl.kernel,
        out_type=(jax.ShapeDtypeStruct((UPK // 128, 128), jnp.float32),
                  jax.ShapeDtypeStruct((IPK // 128, 128), jnp.float32)),
        mesh=mesh,
        compiler_params=pltpu.CompilerParams(needs_layout_passes=False),
        scratch_types=[
            pltpu.VMEM((2, GROWS, 128), jnp.float32),
            pltpu.SemaphoreType.DMA,
            pltpu.SemaphoreType.DMA,
        ],
    )(_repack_body)
    utail = jnp.pad(user_factors[NGU * G * 128:].T,
                    ((0, 0), (0, G * 128 - (NU - NGU * G * 128))))
    itail = jnp.pad(item_factors[NGI * G * 128:].T,
                    ((0, 0), (0, G * 128 - (NI - NGI * G * 128))))
    upk2, ipk2 = repack(user_factors.T, item_factors.T, utail, itail)
    upk = upk2.reshape(UPK)
    ipk = ipk2.reshape(IPK)

    gather = functools.partial(
        pl.kernel,
        out_type=jax.ShapeDtypeStruct((BATCH,), jnp.float32),
        mesh=mesh,
        compiler_params=pltpu.CompilerParams(
            needs_layout_passes=False, use_tc_tiling_on_sc=False),
        scratch_types=[
            pltpu.VMEM((NCHUNK, CHUNK), jnp.int32),
            pltpu.VMEM((NCHUNK, CHUNK), jnp.int32),
            pltpu.VMEM((F, NCHUNK, CHUNK), jnp.int32),
            pltpu.VMEM((F, NCHUNK, CHUNK), jnp.int32),
            pltpu.VMEM((F, BPW), jnp.float32),
            pltpu.VMEM((F, BPW), jnp.float32),
            pltpu.VMEM((BPW,), jnp.float32),
            pltpu.SemaphoreType.DMA,
        ],
    )(_gather_body)
    return gather(user, item, upk, ipk)


def kernel(user, item, user_factors, item_factors):
    return _mf_scores(user.astype(jnp.int32), item.astype(jnp.int32),
                      user_factors, item_factors)
